# Initial kernel scaffold; baseline (speedup 1.0000x reference)
import jax
import jax.numpy as jnp
from jax.experimental import pallas as pl


def kernel(x, edge_index, W_self, W_neigh, b):
    raise NotImplementedError("write your pallas kernel here")


# speedup vs baseline: 5.4497x; 5.4497x over previous
import functools

import jax
import jax.numpy as jnp
from jax import lax
from jax.experimental import pallas as pl
from jax.experimental.pallas import tpu as pltpu
from jax.experimental.pallas import tpu_sc as plsc

N = 10000
E = 320000
D = 128
DP = 144
NC = 2
NS = 16
NW = NC * NS
EPW = E // NW
CHUNK = 80
NCHUNK = EPW // CHUNK
NPAD = 10240
ZROWS = NPAD // NS


def _make_sc_kernel():
    mesh = plsc.VectorSubcoreMesh(core_axis_name="c", subcore_axis_name="s")

    @functools.partial(
        pl.kernel,
        out_type=jax.ShapeDtypeStruct((NC, NPAD, DP), jnp.float32),
        mesh=mesh,
        compiler_params=pltpu.CompilerParams(use_tc_tiling_on_sc=False),
        scratch_types=[
            pltpu.VMEM_SHARED((NPAD, DP), jnp.float32),
            pltpu.VMEM((CHUNK,), jnp.int32),
            pltpu.VMEM((CHUNK,), jnp.int32),
            pltpu.VMEM((CHUNK, DP), jnp.float32),
            pltpu.SemaphoreType.DMA,
        ],
    )
    def sc_kernel(xa_hbm, src_hbm, dst_hbm, zero_hbm, out_hbm,
                  acc, idx_s, idx_d, rows, sem):
        core = lax.axis_index("c")
        sub = lax.axis_index("s")
        wid = core * NS + sub
        ebase = wid * EPW

        pltpu.sync_copy(zero_hbm, acc.at[pl.ds(sub * ZROWS, ZROWS)])
        plsc.subcore_barrier()

        def chunk_body(c, carry):
            off = ebase + c * CHUNK
            pltpu.sync_copy(src_hbm.at[pl.ds(off, CHUNK)], idx_s)
            pltpu.sync_copy(dst_hbm.at[pl.ds(off, CHUNK)], idx_d)
            pltpu.async_copy(xa_hbm.at[idx_s], rows, sem).wait()
            pltpu.sync_copy(rows, acc.at[idx_d], add=True)
            return carry

        lax.fori_loop(0, NCHUNK, chunk_body, 0)
        plsc.subcore_barrier()

        pltpu.sync_copy(acc.at[pl.ds(sub * ZROWS, ZROWS)],
                        out_hbm.at[core, pl.ds(sub * ZROWS, ZROWS)])

    return sc_kernel


_BN = 1000


def _tc_body(x_ref, p0_ref, p1_ref, ws_ref, wn_ref, b_ref, o_ref):
    x = x_ref[...]
    s = p0_ref[...] + p1_ref[...]
    agg = s[:, :D]
    deg = s[:, D:D + 1]
    a = (agg + x) / (deg + 1.0)
    out = (jnp.dot(x, ws_ref[...], preferred_element_type=jnp.float32)
           + jnp.dot(a, wn_ref[...], preferred_element_type=jnp.float32)
           + b_ref[...])
    o_ref[...] = jnp.where(out >= 0, out, 0.01 * out)


def _tc_call(x, p0, p1, W_self, W_neigh, b2d):
    grid = (N // _BN,)
    return pl.pallas_call(
        _tc_body,
        grid=grid,
        in_specs=[
            pl.BlockSpec((_BN, D), lambda i: (i, 0)),
            pl.BlockSpec((_BN, DP), lambda i: (i, 0)),
            pl.BlockSpec((_BN, DP), lambda i: (i, 0)),
            pl.BlockSpec((D, D), lambda i: (0, 0)),
            pl.BlockSpec((D, D), lambda i: (0, 0)),
            pl.BlockSpec((1, D), lambda i: (0, 0)),
        ],
        out_specs=pl.BlockSpec((_BN, D), lambda i: (i, 0)),
        out_shape=jax.ShapeDtypeStruct((N, D), jnp.float32),
    )(x, p0, p1, W_self, W_neigh, b2d)


def kernel(x, edge_index, W_self, W_neigh, b):
    ei = edge_index.astype(jnp.int32)
    src = ei[0]
    dst = ei[1]
    xa = jnp.concatenate(
        [x,
         jnp.ones((N, 1), jnp.float32),
         jnp.zeros((N, DP - D - 1), jnp.float32)], axis=1)
    zero = jnp.zeros((ZROWS, DP), jnp.float32)
    partials = _make_sc_kernel()(xa, src, dst, zero)
    return _tc_call(x, partials[0, :N], partials[1, :N], W_self, W_neigh,
                    b.reshape(1, D))
